# Initial kernel scaffold; baseline (speedup 1.0000x reference)
import jax
import jax.numpy as jnp
from jax.experimental import pallas as pl


def kernel(graph, feat, W, b):
    raise NotImplementedError("write your pallas kernel here")


# speedup vs baseline: 6.0948x; 6.0948x over previous
import jax
import jax.numpy as jnp
from jax import lax
from jax.experimental import pallas as pl
from jax.experimental.pallas import tpu as pltpu
from jax.experimental.pallas import tpu_sc as plsc

N = 10000
E = 320000
FEAT = 128

NP = 10240

NC = 2
NS = 16
NW = NC * NS

EPW = E // NW
CH = 80
NCHUNK = EPW // CH

ROWS_PER_TILE = NP // NS
ROW_CH = 128
NROW_CH = ROWS_PER_TILE // ROW_CH


def _sc_body(src_hbm, dst_hbm, feat_hbm, agg_out, deg_out,
             shared_agg, shared_deg, sidx, didx, rows, zbuf, zdeg, ones_v,
             sem):
    cid = lax.axis_index("c")
    tid = lax.axis_index("s")
    wid = cid * NS + tid

    z16 = jnp.zeros((16,), jnp.float32)
    o16 = jnp.ones((16,), jnp.float32)

    def _zero_row(i, carry):
        for c in range(FEAT // 16):
            zbuf[i, pl.ds(c * 16, 16)] = z16
        return carry

    lax.fori_loop(0, ROW_CH, _zero_row, 0)

    def _zero_deg(i, carry):
        zdeg[pl.ds(i * 16, 16)] = z16
        return carry

    lax.fori_loop(0, ROWS_PER_TILE // 16, _zero_deg, 0)

    for c in range(CH // 16):
        ones_v[pl.ds(c * 16, 16)] = o16

    for k in range(NROW_CH):
        r0 = tid * ROWS_PER_TILE + k * ROW_CH
        pltpu.sync_copy(zbuf, shared_agg.at[pl.ds(r0, ROW_CH), :])
    pltpu.sync_copy(zdeg, shared_deg.at[pl.ds(tid * ROWS_PER_TILE,
                                              ROWS_PER_TILE)])
    plsc.subcore_barrier()

    def _chunk(j, carry):
        base = pl.multiple_of(wid * EPW + j * CH, 8)
        pltpu.sync_copy(src_hbm.at[pl.ds(base, CH)], sidx)
        pltpu.sync_copy(dst_hbm.at[pl.ds(base, CH)], didx)
        pltpu.async_copy(feat_hbm.at[sidx], rows, sem).wait()
        pltpu.sync_copy(rows, shared_agg.at[didx], add=True)
        pltpu.sync_copy(ones_v, shared_deg.at[didx], add=True)
        return carry

    lax.fori_loop(0, NCHUNK, _chunk, 0)
    plsc.subcore_barrier()

    for k in range(NROW_CH):
        r0 = tid * ROWS_PER_TILE + k * ROW_CH
        pltpu.sync_copy(shared_agg.at[pl.ds(r0, ROW_CH), :], zbuf)
        pltpu.sync_copy(zbuf, agg_out.at[cid, pl.ds(r0, ROW_CH), :])
    d0 = tid * ROWS_PER_TILE
    pltpu.sync_copy(shared_deg.at[pl.ds(d0, ROWS_PER_TILE)], zdeg)
    pltpu.sync_copy(zdeg, deg_out.at[cid, pl.ds(d0, ROWS_PER_TILE)])


@jax.jit
def _sc_aggregate(src, dst, feat):
    mesh = plsc.VectorSubcoreMesh(core_axis_name="c", subcore_axis_name="s",
                                  num_cores=NC, num_subcores=NS)
    return pl.kernel(
        _sc_body,
        out_type=[
            jax.ShapeDtypeStruct((NC, NP, FEAT), jnp.float32),
            jax.ShapeDtypeStruct((NC, NP), jnp.float32),
        ],
        mesh=mesh,
        scratch_types=[
            pltpu.VMEM_SHARED((NP, FEAT), jnp.float32),
            pltpu.VMEM_SHARED((NP,), jnp.float32),
            pltpu.VMEM((CH,), jnp.int32),
            pltpu.VMEM((CH,), jnp.int32),
            pltpu.VMEM((CH, FEAT), jnp.float32),
            pltpu.VMEM((ROW_CH, FEAT), jnp.float32),
            pltpu.VMEM((ROWS_PER_TILE,), jnp.float32),
            pltpu.VMEM((CH,), jnp.float32),
            pltpu.SemaphoreType.DMA,
        ],
    )(src, dst, feat)


TC_R = 512


def _tc_body(agg_ref, deg_ref, w_ref, b_ref, out_ref):
    a = agg_ref[0] + agg_ref[1]
    d = deg_ref[0] + deg_ref[1]
    scale = 1.0 / jnp.maximum(d, 1.0)
    a = a * scale
    out_ref[...] = (
        jnp.dot(a, w_ref[...], preferred_element_type=jnp.float32)
        + b_ref[...]
    )


@jax.jit
def _tc_finish(agg_p, deg3, W, b2):
    grid = NP // TC_R
    return pl.pallas_call(
        _tc_body,
        grid=(grid,),
        in_specs=[
            pl.BlockSpec((NC, TC_R, FEAT), lambda i: (0, i, 0)),
            pl.BlockSpec((NC, TC_R, 1), lambda i: (0, i, 0)),
            pl.BlockSpec((FEAT, FEAT), lambda i: (0, 0)),
            pl.BlockSpec((1, FEAT), lambda i: (0, 0)),
        ],
        out_specs=pl.BlockSpec((TC_R, FEAT), lambda i: (i, 0)),
        out_shape=jax.ShapeDtypeStruct((NP, FEAT), jnp.float32),
    )(agg_p, deg3, W, b2)


def kernel(graph, feat, W, b):
    src = graph[0]
    dst = graph[1]
    agg_p, deg_p = _sc_aggregate(src, dst, feat)
    out = _tc_finish(agg_p, deg_p.reshape(NC, NP, 1), W,
                     b.reshape(1, FEAT))
    return out[:N]


# speedup vs baseline: 12.5713x; 2.0626x over previous
import jax
import jax.numpy as jnp
from jax import lax
from jax.experimental import pallas as pl
from jax.experimental.pallas import tpu as pltpu
from jax.experimental.pallas import tpu_sc as plsc

N = 10000
E = 320000
FEAT = 128

NC = 2
NS = 16
NW = NC * NS

EPW = E // NW
CH = 80
NCHUNK = EPW // CH

DEG_PAD = 10240
DEG_PER_TILE = DEG_PAD // NS

AGG_PER_TILE = 640
NROW_CH = AGG_PER_TILE // CH


def _sc_body(src_hbm, dst_hbm, feat_hbm, agg_out, deg_out,
             shared_agg, shared_deg, sidx_all, didx_all, rows0, rows1,
             zdeg, ones_v, sem0, sem1):
    cid = lax.axis_index("c")
    tid = lax.axis_index("s")
    wid = cid * NS + tid

    z16 = jnp.zeros((16,), jnp.float32)
    o16 = jnp.ones((16,), jnp.float32)

    pltpu.sync_copy(src_hbm.at[pl.ds(wid * EPW, EPW)], sidx_all)
    pltpu.sync_copy(dst_hbm.at[wid], didx_all)

    def _zero_row(i, carry):
        for c in range(FEAT // 16):
            rows0[i, pl.ds(c * 16, 16)] = z16
        return carry

    lax.fori_loop(0, CH, _zero_row, 0)

    def _zero_deg(i, carry):
        zdeg[pl.ds(i * 16, 16)] = z16
        return carry

    lax.fori_loop(0, DEG_PER_TILE // 16, _zero_deg, 0)

    for c in range(CH // 16):
        ones_v[pl.ds(c * 16, 16)] = o16

    for k in range(NROW_CH):
        a0 = tid * AGG_PER_TILE + k * CH

        @pl.when(a0 < N)
        def _():
            pltpu.sync_copy(rows0, shared_agg.at[pl.ds(a0, CH), :])
    pltpu.sync_copy(zdeg, shared_deg.at[pl.ds(tid * DEG_PER_TILE,
                                              DEG_PER_TILE)])
    plsc.subcore_barrier()

    def _gstart(j, buf, sem):
        pltpu.async_copy(feat_hbm.at[sidx_all.at[pl.ds(j * CH, CH)]],
                         buf, sem)

    def _gwait(buf, sem):
        pltpu.make_async_copy(feat_hbm.at[pl.ds(0, CH)], buf, sem).wait()

    def _consume(j, buf):
        pltpu.sync_copy(buf, shared_agg.at[didx_all.at[j]], add=True)
        pltpu.sync_copy(ones_v, shared_deg.at[didx_all.at[j]], add=True)

    _gstart(0, rows0, sem0)

    def _pair(j2, carry):
        j = j2 * 2
        _gstart(j + 1, rows1, sem1)
        _gwait(rows0, sem0)
        _consume(j, rows0)

        @pl.when(j + 2 < NCHUNK)
        def _():
            _gstart(j + 2, rows0, sem0)

        _gwait(rows1, sem1)
        _consume(j + 1, rows1)
        return carry

    lax.fori_loop(0, NCHUNK // 2, _pair, 0)
    _gwait(rows0, sem0)
    _consume(NCHUNK - 1, rows0)
    plsc.subcore_barrier()

    for k in range(NROW_CH):
        a0 = tid * AGG_PER_TILE + k * CH

        @pl.when(a0 < N)
        def _():
            pltpu.sync_copy(shared_agg.at[pl.ds(a0, CH), :], rows0)
            pltpu.sync_copy(rows0, agg_out.at[cid, pl.ds(a0, CH), :])
    d0 = tid * DEG_PER_TILE
    pltpu.sync_copy(shared_deg.at[pl.ds(d0, DEG_PER_TILE)], zdeg)
    pltpu.sync_copy(zdeg, deg_out.at[cid, pl.ds(d0, DEG_PER_TILE)])


@jax.jit
def _sc_aggregate(src, dst, feat):
    mesh = plsc.VectorSubcoreMesh(core_axis_name="c", subcore_axis_name="s",
                                  num_cores=NC, num_subcores=NS)
    return pl.kernel(
        _sc_body,
        out_type=[
            jax.ShapeDtypeStruct((NC, N, FEAT), jnp.float32),
            jax.ShapeDtypeStruct((NC, DEG_PAD), jnp.float32),
        ],
        mesh=mesh,
        scratch_types=[
            pltpu.VMEM_SHARED((N, FEAT), jnp.float32),
            pltpu.VMEM_SHARED((DEG_PAD,), jnp.float32),
            pltpu.VMEM((EPW,), jnp.int32),
            pltpu.VMEM((NCHUNK, CH), jnp.int32),
            pltpu.VMEM((CH, FEAT), jnp.float32),
            pltpu.VMEM((CH, FEAT), jnp.float32),
            pltpu.VMEM((DEG_PER_TILE,), jnp.float32),
            pltpu.VMEM((CH,), jnp.float32),
            pltpu.SemaphoreType.DMA,
            pltpu.SemaphoreType.DMA,
        ],
    )(src, dst, feat)


TC_R = 400


def _tc_body(agg_ref, deg_ref, w_ref, b_ref, out_ref):
    a = agg_ref[0] + agg_ref[1]
    d = deg_ref[0] + deg_ref[1]
    scale = 1.0 / jnp.maximum(d, 1.0)
    a = a * scale
    out_ref[...] = (
        jnp.dot(a, w_ref[...], preferred_element_type=jnp.float32)
        + b_ref[...]
    )


@jax.jit
def _tc_finish(agg_p, deg3, W, b2):
    grid = N // TC_R
    return pl.pallas_call(
        _tc_body,
        grid=(grid,),
        in_specs=[
            pl.BlockSpec((NC, TC_R, FEAT), lambda i: (0, i, 0)),
            pl.BlockSpec((NC, TC_R, 1), lambda i: (0, i, 0)),
            pl.BlockSpec((FEAT, FEAT), lambda i: (0, 0)),
            pl.BlockSpec((1, FEAT), lambda i: (0, 0)),
        ],
        out_specs=pl.BlockSpec((TC_R, FEAT), lambda i: (i, 0)),
        out_shape=jax.ShapeDtypeStruct((N, FEAT), jnp.float32),
    )(agg_p, deg3, W, b2)


def kernel(graph, feat, W, b):
    src = graph[0]
    dst = graph[1].reshape(NW, NCHUNK, CH)
    agg_p, deg_p = _sc_aggregate(src, dst, feat)
    return _tc_finish(agg_p, deg_p[:, :N].reshape(NC, N, 1), W,
                      b.reshape(1, FEAT))


# speedup vs baseline: 15.1770x; 1.2073x over previous
import jax
import jax.numpy as jnp
from jax import lax
from jax.experimental import pallas as pl
from jax.experimental.pallas import tpu as pltpu
from jax.experimental.pallas import tpu_sc as plsc

N = 10000
E = 320000
FEAT = 128

NC = 2
NS = 16
NW = NC * NS

EPW = E // NW
CH = 80
NCHUNK = EPW // CH
NBUF = 3

DEG_PAD = 10240
DEG_PER_TILE = DEG_PAD // NS

AGG_PER_TILE = 640
NROW_CH = AGG_PER_TILE // CH

SRC_MASK = (1 << 14) - 1


def _sc_body(packed_hbm, feat_hbm, agg_out, deg_out,
             shared_agg, shared_deg, packed_all,
             rows_a, rows_b, rows_c, sidx_a, sidx_b, sidx_c,
             didx_a, didx_b, didx_c, zdeg, ones_v,
             semg_a, semg_b, semg_c, semsc_a, semsc_b, semsc_c, semw):
    cid = lax.axis_index("c")
    tid = lax.axis_index("s")
    wid = cid * NS + tid

    rows = (rows_a, rows_b, rows_c)
    sidx = (sidx_a, sidx_b, sidx_c)
    didx = (didx_a, didx_b, didx_c)
    semg = (semg_a, semg_b, semg_c)
    semsc = (semsc_a, semsc_b, semsc_c)

    z16 = jnp.zeros((16,), jnp.float32)
    o16 = jnp.ones((16,), jnp.float32)

    pltpu.async_copy(packed_hbm.at[wid], packed_all, semg_a)

    def _zero_row(i, carry):
        for c in range(FEAT // 16):
            rows_a[i, pl.ds(c * 16, 16)] = z16
        return carry

    lax.fori_loop(0, CH, _zero_row, 0)

    def _zero_deg(i, carry):
        zdeg[pl.ds(i * 16, 16)] = z16
        return carry

    lax.fori_loop(0, DEG_PER_TILE // 16, _zero_deg, 0)

    for c in range(CH // 16):
        ones_v[pl.ds(c * 16, 16)] = o16

    for k in range(NROW_CH):
        a0 = tid * AGG_PER_TILE + k * CH

        @pl.when(a0 < N)
        def _():
            pltpu.async_copy(rows_a, shared_agg.at[pl.ds(a0, CH), :], semw)
    pltpu.sync_copy(zdeg, shared_deg.at[pl.ds(tid * DEG_PER_TILE,
                                              DEG_PER_TILE)])
    for k in range(NROW_CH):
        a0 = tid * AGG_PER_TILE + k * CH

        @pl.when(a0 < N)
        def _():
            pltpu.make_async_copy(feat_hbm.at[pl.ds(0, CH)], rows_a,
                                  semw).wait()

    pltpu.make_async_copy(packed_hbm.at[wid], packed_all, semg_a).wait()

    def _unpack(jn, b):
        for c in range(CH // 16):
            v = packed_all[jn, pl.ds(c * 16, 16)]
            sidx[b][pl.ds(c * 16, 16)] = v & SRC_MASK
            didx[b][pl.ds(c * 16, 16)] = v >> 14

    def _gstart(b):
        pltpu.async_copy(feat_hbm.at[sidx[b]], rows[b], semg[b])

    def _gwait(b):
        pltpu.make_async_copy(feat_hbm.at[pl.ds(0, CH)], rows[b],
                              semg[b]).wait()

    def _scstart(b):
        pltpu.async_copy(rows[b], shared_agg.at[didx[b]], semsc[b],
                         add=True)

    def _scwait(b):
        pltpu.make_async_copy(feat_hbm.at[pl.ds(0, CH)], rows[b],
                              semsc[b]).wait()

    for b in range(NBUF):
        _unpack(b, b)
        _gstart(b)

    plsc.subcore_barrier()

    def _step(j3, carry):
        for b in range(NBUF):
            j = j3 * NBUF + b
            _gwait(b)
            _scstart(b)
            pltpu.sync_copy(ones_v, shared_deg.at[didx[b]], add=True)
            _scwait(b)
            jn = j + NBUF

            @pl.when(jn < NCHUNK)
            def _():
                _unpack(jn, b)
                _gstart(b)
        return carry

    lax.fori_loop(0, (NCHUNK + NBUF - 1) // NBUF - 1, _step, 0)
    for b in range(NCHUNK % NBUF):
        _gwait(b)
        _scstart(b)
        pltpu.sync_copy(ones_v, shared_deg.at[didx[b]], add=True)
        _scwait(b)
    plsc.subcore_barrier()

    for k in range(NROW_CH):
        a0 = tid * AGG_PER_TILE + k * CH
        b = k % NBUF

        @pl.when(a0 < N)
        def _():
            if k >= NBUF:
                pltpu.make_async_copy(feat_hbm.at[pl.ds(0, CH)], rows[b],
                                      semw).wait()
            pltpu.sync_copy(shared_agg.at[pl.ds(a0, CH), :], rows[b])
            pltpu.async_copy(rows[b], agg_out.at[cid, pl.ds(a0, CH), :],
                             semw)
    for _ in range(NBUF):
        pltpu.make_async_copy(feat_hbm.at[pl.ds(0, CH)], rows_a,
                              semw).wait()
    d0 = tid * DEG_PER_TILE
    pltpu.sync_copy(shared_deg.at[pl.ds(d0, DEG_PER_TILE)], zdeg)
    pltpu.sync_copy(zdeg, deg_out.at[cid, pl.ds(d0, DEG_PER_TILE)])


@jax.jit
def _sc_aggregate(packed, feat):
    mesh = plsc.VectorSubcoreMesh(core_axis_name="c", subcore_axis_name="s",
                                  num_cores=NC, num_subcores=NS)
    return pl.kernel(
        _sc_body,
        out_type=[
            jax.ShapeDtypeStruct((NC, N, FEAT), jnp.float32),
            jax.ShapeDtypeStruct((NC, DEG_PAD), jnp.float32),
        ],
        mesh=mesh,
        scratch_types=[
            pltpu.VMEM_SHARED((N, FEAT), jnp.float32),
            pltpu.VMEM_SHARED((DEG_PAD,), jnp.float32),
            pltpu.VMEM((NCHUNK, CH), jnp.int32),
            pltpu.VMEM((CH, FEAT), jnp.float32),
            pltpu.VMEM((CH, FEAT), jnp.float32),
            pltpu.VMEM((CH, FEAT), jnp.float32),
            pltpu.VMEM((CH,), jnp.int32),
            pltpu.VMEM((CH,), jnp.int32),
            pltpu.VMEM((CH,), jnp.int32),
            pltpu.VMEM((CH,), jnp.int32),
            pltpu.VMEM((CH,), jnp.int32),
            pltpu.VMEM((CH,), jnp.int32),
            pltpu.VMEM((DEG_PER_TILE,), jnp.float32),
            pltpu.VMEM((CH,), jnp.float32),
            pltpu.SemaphoreType.DMA,
            pltpu.SemaphoreType.DMA,
            pltpu.SemaphoreType.DMA,
            pltpu.SemaphoreType.DMA,
            pltpu.SemaphoreType.DMA,
            pltpu.SemaphoreType.DMA,
            pltpu.SemaphoreType.DMA,
        ],
    )(packed, feat)


TC_R = 400


def _tc_body(agg_ref, deg_ref, w_ref, b_ref, out_ref):
    a = agg_ref[0] + agg_ref[1]
    d = deg_ref[0] + deg_ref[1]
    scale = 1.0 / jnp.maximum(d, 1.0)
    a = a * scale
    out_ref[...] = (
        jnp.dot(a, w_ref[...], preferred_element_type=jnp.float32)
        + b_ref[...]
    )


@jax.jit
def _tc_finish(agg_p, deg3, W, b2):
    grid = N // TC_R
    return pl.pallas_call(
        _tc_body,
        grid=(grid,),
        in_specs=[
            pl.BlockSpec((NC, TC_R, FEAT), lambda i: (0, i, 0)),
            pl.BlockSpec((NC, TC_R, 1), lambda i: (0, i, 0)),
            pl.BlockSpec((FEAT, FEAT), lambda i: (0, 0)),
            pl.BlockSpec((1, FEAT), lambda i: (0, 0)),
        ],
        out_specs=pl.BlockSpec((TC_R, FEAT), lambda i: (i, 0)),
        out_shape=jax.ShapeDtypeStruct((N, FEAT), jnp.float32),
    )(agg_p, deg3, W, b2)


def kernel(graph, feat, W, b):
    packed = (graph[0] | (graph[1] << 14)).reshape(NW, NCHUNK, CH)
    agg_p, deg_p = _sc_aggregate(packed, feat)
    return _tc_finish(agg_p, deg_p[:, :N].reshape(NC, N, 1), W,
                      b.reshape(1, FEAT))


# speedup vs baseline: 15.3001x; 1.0081x over previous
import jax
import jax.numpy as jnp
from jax import lax
from jax.experimental import pallas as pl
from jax.experimental.pallas import tpu as pltpu
from jax.experimental.pallas import tpu_sc as plsc

N = 10000
E = 320000
FEAT = 128

NC = 2
NS = 16
NW = NC * NS

EPW = E // NW
CH = 80
NCHUNK = EPW // CH
NBUF = 4

DEG_PAD = 10240
DEG_PER_TILE = DEG_PAD // NS

AGG_PER_TILE = 640
NROW_CH = AGG_PER_TILE // CH

SRC_MASK = (1 << 14) - 1

STEADY = (NCHUNK // (2 * NBUF)) * (2 * NBUF)


def _sc_body(packed_hbm, feat_hbm, agg_out, deg_out,
             shared_agg, shared_deg,
             rows_a, rows_b, rows_c, rows_d,
             pk_a, pk_b, pk_c, pk_d,
             sidx_a, sidx_b, sidx_c, sidx_d,
             didx_a, didx_b, didx_c, didx_d,
             zdeg, ones_v,
             semi_a0, semi_a1, semi_b0, semi_b1, semi_c0, semi_c1,
             semi_d0, semi_d1,
             semg_a, semg_b, semg_c, semg_d,
             semsc_a, semsc_b, semsc_c, semsc_d, semw):
    cid = lax.axis_index("c")
    tid = lax.axis_index("s")
    wid = cid * NS + tid

    rows = (rows_a, rows_b, rows_c, rows_d)
    pk = (pk_a, pk_b, pk_c, pk_d)
    sidx = (sidx_a, sidx_b, sidx_c, sidx_d)
    didx = (didx_a, didx_b, didx_c, didx_d)
    semi = ((semi_a0, semi_a1), (semi_b0, semi_b1),
            (semi_c0, semi_c1), (semi_d0, semi_d1))
    semg = (semg_a, semg_b, semg_c, semg_d)
    semsc = (semsc_a, semsc_b, semsc_c, semsc_d)

    z16 = jnp.zeros((16,), jnp.float32)
    o16 = jnp.ones((16,), jnp.float32)

    def _istart(jn, b, p):
        pltpu.async_copy(packed_hbm.at[wid, jn], pk[b].at[p], semi[b][p])

    def _iwait(b, p):
        pltpu.make_async_copy(packed_hbm.at[wid, 0], pk[b].at[p],
                              semi[b][p]).wait()

    def _unpack(b, p):
        for c in range(CH // 16):
            v = pk[b][p, pl.ds(c * 16, 16)]
            sidx[b][p, pl.ds(c * 16, 16)] = v & SRC_MASK
            didx[b][p, pl.ds(c * 16, 16)] = v >> 14

    def _gstart(b, p):
        pltpu.async_copy(feat_hbm.at[sidx[b].at[p]], rows[b], semg[b])

    def _gwait(b):
        pltpu.make_async_copy(feat_hbm.at[pl.ds(0, CH)], rows[b],
                              semg[b]).wait()

    def _scstart(b, p):
        pltpu.async_copy(rows[b], shared_agg.at[didx[b].at[p]], semsc[b],
                         add=True)

    def _scwait(b):
        pltpu.make_async_copy(feat_hbm.at[pl.ds(0, CH)], rows[b],
                              semsc[b]).wait()

    for j in range(2 * NBUF):
        _istart(j, j % NBUF, j // NBUF)

    def _zero_row(i, carry):
        for c in range(FEAT // 16):
            rows_a[i, pl.ds(c * 16, 16)] = z16
        return carry

    lax.fori_loop(0, CH, _zero_row, 0)

    def _zero_deg(i, carry):
        zdeg[pl.ds(i * 16, 16)] = z16
        return carry

    lax.fori_loop(0, DEG_PER_TILE // 16, _zero_deg, 0)

    for c in range(CH // 16):
        ones_v[pl.ds(c * 16, 16)] = o16

    for k in range(NROW_CH):
        a0 = tid * AGG_PER_TILE + k * CH

        @pl.when(a0 < N)
        def _():
            pltpu.async_copy(rows_a, shared_agg.at[pl.ds(a0, CH), :], semw)
    pltpu.sync_copy(zdeg, shared_deg.at[pl.ds(tid * DEG_PER_TILE,
                                              DEG_PER_TILE)])
    for k in range(NROW_CH):
        a0 = tid * AGG_PER_TILE + k * CH

        @pl.when(a0 < N)
        def _():
            pltpu.make_async_copy(feat_hbm.at[pl.ds(0, CH)], rows_a,
                                  semw).wait()

    for b in range(NBUF):
        _iwait(b, 0)
        _unpack(b, 0)
        _gstart(b, 0)

    plsc.subcore_barrier()

    def _consume(b, p):
        _gwait(b)
        _scstart(b, p)
        pltpu.sync_copy(ones_v, shared_deg.at[didx[b].at[p]], add=True)
        _scwait(b)

    def _step(j8, carry):
        for q in range(2 * NBUF):
            j = j8 * (2 * NBUF) + q
            b = q % NBUF
            p = q // NBUF
            _consume(b, p)
            jn = j + NBUF

            @pl.when(jn < NCHUNK)
            def _():
                _iwait(b, 1 - p)
                _unpack(b, 1 - p)
                _gstart(b, 1 - p)

            jn2 = j + 2 * NBUF

            @pl.when(jn2 < NCHUNK)
            def _():
                _istart(jn2, b, p)
        return carry

    lax.fori_loop(0, STEADY // (2 * NBUF), _step, 0)
    for j in range(STEADY, NCHUNK):
        b = j % NBUF
        p = (j // NBUF) % 2
        _consume(b, p)
        jn = j + NBUF
        if jn < NCHUNK:
            _iwait(jn % NBUF, (jn // NBUF) % 2)
            _unpack(jn % NBUF, (jn // NBUF) % 2)
            _gstart(jn % NBUF, (jn // NBUF) % 2)
    plsc.subcore_barrier()

    for k in range(NROW_CH):
        a0 = tid * AGG_PER_TILE + k * CH
        b = k % NBUF

        @pl.when(a0 < N)
        def _():
            if k >= NBUF:
                pltpu.make_async_copy(feat_hbm.at[pl.ds(0, CH)], rows[b],
                                      semw).wait()
            pltpu.sync_copy(shared_agg.at[pl.ds(a0, CH), :], rows[b])
            pltpu.async_copy(rows[b], agg_out.at[cid, pl.ds(a0, CH), :],
                             semw)
    for _ in range(NBUF):
        pltpu.make_async_copy(feat_hbm.at[pl.ds(0, CH)], rows_a,
                              semw).wait()
    d0 = tid * DEG_PER_TILE
    pltpu.sync_copy(shared_deg.at[pl.ds(d0, DEG_PER_TILE)], zdeg)
    pltpu.sync_copy(zdeg, deg_out.at[cid, pl.ds(d0, DEG_PER_TILE)])


@jax.jit
def _sc_aggregate(packed, feat):
    mesh = plsc.VectorSubcoreMesh(core_axis_name="c", subcore_axis_name="s",
                                  num_cores=NC, num_subcores=NS)
    idx2 = pltpu.VMEM((2, CH), jnp.int32)
    rowbuf = pltpu.VMEM((CH, FEAT), jnp.float32)
    return pl.kernel(
        _sc_body,
        out_type=[
            jax.ShapeDtypeStruct((NC, N, FEAT), jnp.float32),
            jax.ShapeDtypeStruct((NC, DEG_PAD), jnp.float32),
        ],
        mesh=mesh,
        scratch_types=(
            [pltpu.VMEM_SHARED((N, FEAT), jnp.float32),
             pltpu.VMEM_SHARED((DEG_PAD,), jnp.float32)]
            + [rowbuf] * NBUF
            + [idx2] * (3 * NBUF)
            + [pltpu.VMEM((DEG_PER_TILE,), jnp.float32),
               pltpu.VMEM((CH,), jnp.float32)]
            + [pltpu.SemaphoreType.DMA] * (2 * NBUF + NBUF + NBUF + 1)
        ),
    )(packed, feat)


TC_R = 400


def _tc_body(agg_ref, deg_ref, w_ref, b_ref, out_ref):
    a = agg_ref[0] + agg_ref[1]
    d = deg_ref[0] + deg_ref[1]
    scale = 1.0 / jnp.maximum(d, 1.0)
    a = a * scale
    out_ref[...] = (
        jnp.dot(a, w_ref[...], preferred_element_type=jnp.float32)
        + b_ref[...]
    )


@jax.jit
def _tc_finish(agg_p, deg3, W, b2):
    grid = N // TC_R
    return pl.pallas_call(
        _tc_body,
        grid=(grid,),
        in_specs=[
            pl.BlockSpec((NC, TC_R, FEAT), lambda i: (0, i, 0)),
            pl.BlockSpec((NC, TC_R, 1), lambda i: (0, i, 0)),
            pl.BlockSpec((FEAT, FEAT), lambda i: (0, 0)),
            pl.BlockSpec((1, FEAT), lambda i: (0, 0)),
        ],
        out_specs=pl.BlockSpec((TC_R, FEAT), lambda i: (i, 0)),
        out_shape=jax.ShapeDtypeStruct((N, FEAT), jnp.float32),
    )(agg_p, deg3, W, b2)


def kernel(graph, feat, W, b):
    packed = (graph[0] | (graph[1] << 14)).reshape(NW, NCHUNK, CH)
    agg_p, deg_p = _sc_aggregate(packed, feat)
    return _tc_finish(agg_p, deg_p[:, :N].reshape(NC, N, 1), W,
                      b.reshape(1, FEAT))
